# baseline (speedup 1.0000x reference)
import jax
import jax.numpy as jnp
from jax import lax
from jax.experimental import pallas as pl
from jax.experimental.pallas import tpu as pltpu
from jax.experimental.pallas import tpu_sc as plsc

NC = 2
NS = 16
NW = NC * NS
L = 16

F = 26
D = 32
G = 4
GW = G * D
CHUNK = 32
M = CHUNK * F
SW = 64
NSTREAM = M // SW


def _body(inputs_hbm, table_hbm, offt_hbm, out_hbm,
          in_v, off_v, idx_v, qoff_v, buf_v, out_v, sem):
    wid = lax.axis_index("s") * NC + lax.axis_index("c")
    n_chunks = out_hbm.shape[0] // (NW * CHUNK)

    pltpu.sync_copy(offt_hbm, off_v)

    def chunk_body(c, carry):
        base = (wid * n_chunks + c) * M
        pltpu.sync_copy(inputs_hbm.at[pl.ds(base, M)], in_v)

        def idx_body(i, carry2):
            s = i * L
            idx = in_v[pl.ds(s, L)] + off_v[pl.ds(s, L)]
            idx_v[pl.ds(s, L)] = idx >> 2
            qoff_v[pl.ds(s, L)] = (idx & 3) * D
            return carry2
        lax.fori_loop(0, M // L, idx_body, 0, unroll=False)

        descs = [
            pltpu.async_copy(table_hbm.at[idx_v.at[pl.ds(j * SW, SW)]],
                             buf_v.at[pl.ds(j * SW, SW)], sem)
            for j in range(NSTREAM)
        ]
        for d in descs:
            d.wait()

        def sum_body(r, carry2):
            g = r * F
            qv0 = qoff_v[pl.ds(g, L)]
            qv1 = qoff_v[pl.ds(g + F - L, L)]
            q0 = qv0[0]
            acc0 = buf_v[g, pl.ds(q0, L)]
            acc1 = buf_v[g, pl.ds(q0 + L, L)]
            for f in range(1, F):
                q = qv0[f] if f < L else qv1[f - (F - L)]
                acc0 = acc0 + buf_v[g + f, pl.ds(q, L)]
                acc1 = acc1 + buf_v[g + f, pl.ds(q + L, L)]
            out_v[r, pl.ds(0, L)] = acc0
            out_v[r, pl.ds(L, L)] = acc1
            return carry2
        lax.fori_loop(0, CHUNK, sum_body, 0, unroll=False)

        pltpu.sync_copy(out_v, out_hbm.at[pl.ds((wid * n_chunks + c) * CHUNK,
                                                CHUNK)])
        return carry

    lax.fori_loop(0, n_chunks, chunk_body, 0, unroll=False)


def kernel(inputs, table, offset):
    B = inputs.shape[0]
    inputs_flat = inputs.reshape(B * F)
    off_tiled = jnp.tile(offset, CHUNK)
    table_g = table.reshape(table.shape[0] // G, GW)

    k = pl.kernel(
        _body,
        out_type=jax.ShapeDtypeStruct((B, D), jnp.float32),
        mesh=plsc.VectorSubcoreMesh(core_axis_name="c", subcore_axis_name="s"),
        scratch_types=[
            pltpu.VMEM((M,), jnp.int32),
            pltpu.VMEM((M,), jnp.int32),
            pltpu.VMEM((M,), jnp.int32),
            pltpu.VMEM((M,), jnp.int32),
            pltpu.VMEM((M, GW), jnp.float32),
            pltpu.VMEM((CHUNK, D), jnp.float32),
            pltpu.SemaphoreType.DMA,
        ],
        compiler_params=pltpu.CompilerParams(use_tc_tiling_on_sc=True,
                                             needs_layout_passes=False),
    )
    return k(inputs_flat, table_g, off_tiled)


# speedup vs baseline: 1.0913x; 1.0913x over previous
import jax
import jax.numpy as jnp
from jax import lax
from jax.experimental import pallas as pl
from jax.experimental.pallas import tpu as pltpu
from jax.experimental.pallas import tpu_sc as plsc

NC = 2
NS = 16
NW = NC * NS
L = 16

F = 26
D = 32
G = 4
GW = G * D
CHUNK = 32
M = CHUNK * F
SW = 64
NSTREAM = M // SW


def _body(inputs_hbm, table_hbm, offt_hbm, out_hbm,
          in_v, off_v, idx_v, qoff_v, buf_v, out_v, sem):
    wid = lax.axis_index("s") * NC + lax.axis_index("c")
    n_chunks = out_hbm.shape[0] // (NW * CHUNK)

    pltpu.sync_copy(offt_hbm, off_v)

    def chunk_body(c, carry):
        base = (wid * n_chunks + c) * M
        pltpu.sync_copy(inputs_hbm.at[pl.ds(base, M)], in_v)

        def idx_body(i, carry2):
            s = i * L
            idx = in_v[pl.ds(s, L)] + off_v[pl.ds(s, L)]
            idx_v[pl.ds(s, L)] = idx >> 2
            qoff_v[pl.ds(s, L)] = (idx & 3) * D
            return carry2
        lax.fori_loop(0, M // L, idx_body, 0, unroll=False)

        descs = [
            pltpu.async_copy(table_hbm.at[idx_v.at[pl.ds(j * SW, SW)]],
                             buf_v.at[pl.ds(j * SW, SW)], sem)
            for j in range(NSTREAM)
        ]
        for d in descs:
            d.wait()

        def sum_body(r, carry2):
            g = r * F
            qv0 = qoff_v[pl.ds(g, L)]
            qv1 = qoff_v[pl.ds(g + F - L, L)]
            q0 = qv0[0]
            acc0 = buf_v[g, pl.ds(q0, L)]
            acc1 = buf_v[g, pl.ds(q0 + L, L)]
            for f in range(1, F):
                q = qv0[f] if f < L else qv1[f - (F - L)]
                acc0 = acc0 + buf_v[g + f, pl.ds(q, L)]
                acc1 = acc1 + buf_v[g + f, pl.ds(q + L, L)]
            out_v[r, pl.ds(0, L)] = acc0
            out_v[r, pl.ds(L, L)] = acc1
            return carry2
        lax.fori_loop(0, CHUNK, sum_body, 0, unroll=False)

        pltpu.sync_copy(out_v, out_hbm.at[pl.ds((wid * n_chunks + c) * CHUNK,
                                                CHUNK)])
        return carry

    lax.fori_loop(0, n_chunks, chunk_body, 0, unroll=False)


def kernel(inputs, table, offset):
    B = inputs.shape[0]
    inputs_flat = inputs.reshape(B * F)
    off_tiled = jnp.tile(offset, CHUNK)
    V = table.shape[0]
    table_g = (table.T.reshape(D, V // G, G)
               .transpose(1, 2, 0)
               .reshape(V // G, GW))

    k = pl.kernel(
        _body,
        out_type=jax.ShapeDtypeStruct((B, D), jnp.float32),
        mesh=plsc.VectorSubcoreMesh(core_axis_name="c", subcore_axis_name="s"),
        scratch_types=[
            pltpu.VMEM((M,), jnp.int32),
            pltpu.VMEM((M,), jnp.int32),
            pltpu.VMEM((M,), jnp.int32),
            pltpu.VMEM((M,), jnp.int32),
            pltpu.VMEM((M, GW), jnp.float32),
            pltpu.VMEM((CHUNK, D), jnp.float32),
            pltpu.SemaphoreType.DMA,
        ],
        compiler_params=pltpu.CompilerParams(use_tc_tiling_on_sc=True,
                                             needs_layout_passes=False),
    )
    return k(inputs_flat, table_g, off_tiled)
